# baseline (speedup 1.0000x reference)
import jax
import jax.numpy as jnp
from jax import lax
from jax.experimental import pallas as pl
from jax.experimental.pallas import tpu as pltpu

_TI = 400
_NBUF = 2
_TX = 2000
_NXBUF = 2


def _body(x_hbm, adj_hbm, w1_ref, w2_ref, wo_ref, bo_ref, out_ref,
          s1, c_acc, h1_acc, xbuf, abuf, xsem, asem):
    n = adj_hbm.shape[0]
    na = n // _TI
    nx = n // _TX

    def start_x(k, slot):
        pltpu.make_async_copy(
            x_hbm.at[pl.ds(k * _TX, _TX), :], xbuf.at[slot], xsem.at[slot]
        ).start()

    def wait_x(slot):
        pltpu.make_async_copy(
            x_hbm.at[pl.ds(0, _TX), :], xbuf.at[slot], xsem.at[slot]
        ).wait()

    def start_a(r, slot):
        pltpu.make_async_copy(
            adj_hbm.at[pl.ds(r * _TI, _TI), :], abuf.at[slot], asem.at[slot]
        ).start()

    def wait_a(slot):
        pltpu.make_async_copy(
            adj_hbm.at[pl.ds(0, _TI), :], abuf.at[slot], asem.at[slot]
        ).wait()

    start_x(0, 0)
    for b in range(_NBUF):
        start_a(b, b)

    def x_chunk(k, b):
        @pl.when(k + 1 < nx)
        def _():
            start_x(k + 1, (b + 1) % _NXBUF)

        wait_x(b)
        s1[pl.ds(k * _TX, _TX), :] = jnp.dot(
            xbuf[b], w1_ref[...], preferred_element_type=jnp.float32)

    def x_group(g, carry):
        for b in range(_NXBUF):
            x_chunk(g * _NXBUF + b, b)
        return carry

    lax.fori_loop(0, nx // _NXBUF, x_group, 0)
    for j in range(nx % _NXBUF):
        x_chunk((nx // _NXBUF) * _NXBUF + j, j)

    c_acc[...] = jnp.zeros_like(c_acc)

    def a_stripe(r, b):
        wait_a(b)
        blk = abuf[b]
        h1 = jnp.maximum(
            jnp.dot(blk, s1[...], preferred_element_type=jnp.float32), 0.0)
        h1_acc[pl.ds(r * _TI, _TI), :] = h1
        c_acc[...] = c_acc[...] + jnp.sum(blk, axis=0, keepdims=True)

        @pl.when(r + _NBUF < na)
        def _():
            start_a(r + _NBUF, b)

    def a_group(g, carry):
        for b in range(_NBUF):
            a_stripe(g * _NBUF + b, b)
        return carry

    lax.fori_loop(0, na // _NBUF, a_group, 0)
    for j in range(na % _NBUF):
        a_stripe((na // _NBUF) * _NBUF + j, j)

    pooled = jnp.dot(c_acc[...], h1_acc[...],
                     preferred_element_type=jnp.float32)
    z = jnp.dot(pooled, w2_ref[...],
                preferred_element_type=jnp.float32)
    o = jnp.dot(z, wo_ref[...],
                preferred_element_type=jnp.float32) + bo_ref[...]
    out_ref[...] = jax.nn.sigmoid(o)


def kernel(x, adj, W1, W2, Wo, bo):
    n, d_in = x.shape
    h1_dim = W1.shape[1]

    out = pl.pallas_call(
        _body,
        in_specs=[
            pl.BlockSpec(memory_space=pl.ANY),
            pl.BlockSpec(memory_space=pl.ANY),
            pl.BlockSpec(memory_space=pltpu.VMEM),
            pl.BlockSpec(memory_space=pltpu.VMEM),
            pl.BlockSpec(memory_space=pltpu.VMEM),
            pl.BlockSpec(memory_space=pltpu.VMEM),
        ],
        out_specs=pl.BlockSpec(memory_space=pltpu.VMEM),
        out_shape=jax.ShapeDtypeStruct((1, 1), jnp.float32),
        scratch_shapes=[
            pltpu.VMEM((n, h1_dim), jnp.float32),
            pltpu.VMEM((1, n), jnp.float32),
            pltpu.VMEM((n, h1_dim), jnp.float32),
            pltpu.VMEM((_NXBUF, _TX, d_in), jnp.float32),
            pltpu.VMEM((_NBUF, _TI, n), jnp.float32),
            pltpu.SemaphoreType.DMA((_NXBUF,)),
            pltpu.SemaphoreType.DMA((_NBUF,)),
        ],
    )(x, adj, W1, W2, Wo, bo.reshape(1, 1))

    return out.reshape(1)


# speedup vs baseline: 1.0565x; 1.0565x over previous
import functools

import jax
import jax.numpy as jnp
from jax.experimental import pallas as pl
from jax.experimental.pallas import tpu as pltpu


def _body(nxblk, x_ref, adj_ref, w1_ref, w2_ref, wo_ref, bo_ref, out_ref,
          s1, c_acc, h1_acc):
    i = pl.program_id(0)
    nsteps = pl.num_programs(0)
    ti = adj_ref.shape[0]
    tx = x_ref.shape[0]

    @pl.when(i < nxblk)
    def _prologue():
        s1[pl.ds(i * tx, tx), :] = jnp.dot(
            x_ref[...], w1_ref[...],
            preferred_element_type=jnp.float32).astype(jnp.bfloat16)

    @pl.when(i >= nxblk)
    def _stream():
        r = i - nxblk
        blk = adj_ref[...]
        h1 = jnp.maximum(
            jnp.dot(blk.astype(jnp.bfloat16), s1[...],
                    preferred_element_type=jnp.float32), 0.0)
        h1_acc[pl.ds(r * ti, ti), :] = h1.astype(jnp.bfloat16)
        colsum = jnp.sum(blk, axis=0, keepdims=True)
        c_acc[...] = jnp.where(r == 0, colsum, c_acc[...] + colsum)

    @pl.when(i == nsteps - 1)
    def _epilogue():
        pooled = jnp.dot(c_acc[...].astype(jnp.bfloat16), h1_acc[...],
                         preferred_element_type=jnp.float32)
        z = jnp.dot(pooled, w2_ref[...],
                    preferred_element_type=jnp.float32)
        o = jnp.dot(z, wo_ref[...],
                    preferred_element_type=jnp.float32) + bo_ref[...]
        out_ref[...] = jax.nn.sigmoid(o)


def kernel(x, adj, W1, W2, Wo, bo):
    n, d_in = x.shape
    h1_dim = W1.shape[1]
    h2_dim = W2.shape[1]

    ti = 400
    tx = 2000
    nblk = n // ti
    nxblk = n // tx

    out = pl.pallas_call(
        functools.partial(_body, nxblk),
        grid=(nblk + nxblk,),
        in_specs=[
            pl.BlockSpec((tx, d_in), lambda i: (jnp.minimum(i, nxblk - 1), 0)),
            pl.BlockSpec((ti, n), lambda i: (jnp.maximum(i - nxblk, 0), 0)),
            pl.BlockSpec((d_in, h1_dim), lambda i: (0, 0)),
            pl.BlockSpec((h1_dim, h2_dim), lambda i: (0, 0)),
            pl.BlockSpec((h2_dim, 1), lambda i: (0, 0)),
            pl.BlockSpec((1, 1), lambda i: (0, 0)),
        ],
        out_specs=pl.BlockSpec((1, 1), lambda i: (0, 0)),
        out_shape=jax.ShapeDtypeStruct((1, 1), jnp.float32),
        scratch_shapes=[
            pltpu.VMEM((n, h1_dim), jnp.bfloat16),
            pltpu.VMEM((1, n), jnp.float32),
            pltpu.VMEM((n, h1_dim), jnp.bfloat16),
        ],
        compiler_params=pltpu.CompilerParams(
            dimension_semantics=("arbitrary",)),
    )(x, adj, W1, W2, Wo, bo.reshape(1, 1))

    return out.reshape(1)
